# baseline (speedup 1.0000x reference)
import functools

import jax
import jax.numpy as jnp
from jax import lax
from jax.experimental import pallas as pl
from jax.experimental.pallas import tpu as pltpu
from jax.experimental.pallas import tpu_sc as plsc

N_NODES = 10000
N_PAD = 10240
N_EDGES = 320000
E_PAD = 327680
D = 128
EPS = 1e-5

_NC = 2
_NS = 16
_CH = 64
_ROWS_PER_W = E_PAD // (_NC * _NS) // _CH
_QROWS = _ROWS_PER_W // 4
_NODE_ROWS_PER_S = N_PAD // _NS
_NBUF = 4


def _sc_body(src_hbm, dst_hbm, x_hbm, zeros2d, zeros1d, ones1d,
             agg_out, deg_out,
             src_v, dst_v, gbufs, ones_v, zd_v, agg_sh, deg_sh,
             gsem0, gsem1, gsem2, gsem3, dsem):
    c = lax.axis_index("c")
    s = lax.axis_index("s")
    wid = c * _NS + s
    gsems = (gsem0, gsem1, gsem2, gsem3)

    pltpu.sync_copy(zeros2d, gbufs.at[0])
    pltpu.sync_copy(zeros1d, zd_v)
    for j in range(_NODE_ROWS_PER_S // _CH):
        pltpu.sync_copy(gbufs.at[0], agg_sh.at[pl.ds(s * _NODE_ROWS_PER_S + j * _CH, _CH)])
    pltpu.sync_copy(zd_v, deg_sh.at[pl.ds(s * _NODE_ROWS_PER_S, _NODE_ROWS_PER_S)])

    base = wid * _ROWS_PER_W
    pltpu.sync_copy(ones1d, ones_v)

    first = True
    for q in range(_ROWS_PER_W // _QROWS):
        pltpu.sync_copy(src_hbm.at[pl.ds(base + q * _QROWS, _QROWS)], src_v)
        pltpu.sync_copy(dst_hbm.at[pl.ds(base + q * _QROWS, _QROWS)], dst_v)

        for b in range(_NBUF):
            pltpu.async_copy(x_hbm.at[src_v.at[b]], gbufs.at[b], gsems[b])

        if first:
            plsc.subcore_barrier()
            first = False

        def outer(i4, _):
            for b in range(_NBUF):
                i = i4 * _NBUF + b
                pltpu.make_async_copy(x_hbm.at[src_v.at[i]], gbufs.at[b], gsems[b]).wait()
                pltpu.sync_copy(gbufs.at[b], agg_sh.at[dst_v.at[i]], add=True)
                pltpu.sync_copy(ones_v, deg_sh.at[dst_v.at[i]], add=True)

                @pl.when(i + _NBUF < _QROWS)
                def _():
                    pltpu.async_copy(x_hbm.at[src_v.at[i + _NBUF]], gbufs.at[b], gsems[b])
            return ()

        lax.fori_loop(0, _QROWS // _NBUF, outer, (), unroll=False)

    plsc.subcore_barrier()

    nbase = s * _NODE_ROWS_PER_S
    for j in range(_NODE_ROWS_PER_S // _CH):
        b = j % _NBUF
        pltpu.sync_copy(agg_sh.at[pl.ds(nbase + j * _CH, _CH)], gbufs.at[b])
        pltpu.sync_copy(gbufs.at[b], agg_out.at[c, pl.ds(nbase + j * _CH, _CH)])
    pltpu.sync_copy(deg_sh.at[pl.ds(nbase, _NODE_ROWS_PER_S)], zd_v)
    pltpu.sync_copy(zd_v, deg_out.at[c, pl.ds(nbase, _NODE_ROWS_PER_S)])


def _make_sc_call():
    return functools.partial(
        pl.kernel,
        mesh=plsc.VectorSubcoreMesh(core_axis_name="c", subcore_axis_name="s"),
        out_type=[
            jax.ShapeDtypeStruct((_NC, N_PAD, D), jnp.float32),
            jax.ShapeDtypeStruct((_NC, N_PAD), jnp.float32),
        ],
        scratch_types=[
            pltpu.VMEM((_QROWS, _CH), jnp.int32),
            pltpu.VMEM((_QROWS, _CH), jnp.int32),
            pltpu.VMEM((_NBUF, _CH, D), jnp.float32),
            pltpu.VMEM((_CH,), jnp.float32),
            pltpu.VMEM((_NODE_ROWS_PER_S,), jnp.float32),
            pltpu.VMEM_SHARED((N_PAD, D), jnp.float32),
            pltpu.VMEM_SHARED((N_PAD,), jnp.float32),
            pltpu.SemaphoreType.DMA,
            pltpu.SemaphoreType.DMA,
            pltpu.SemaphoreType.DMA,
            pltpu.SemaphoreType.DMA,
            pltpu.SemaphoreType.DMA,
        ],
    )(_sc_body)


def _tc_body(a_ref, d_ref, x_ref, wl_ref, bl_ref, wr_ref, g_ref, b_ref, o_ref):
    agg = a_ref[0, :N_NODES, :] + a_ref[1, :N_NODES, :]
    deg = d_ref[0, :N_NODES] + d_ref[1, :N_NODES]
    deg = jnp.maximum(deg, 1.0)
    mean = agg / deg[:, None]
    x = x_ref[...]
    dn = (((1,), (1,)), ((), ()))
    h = lax.dot_general(mean, wl_ref[...], dn,
                        precision=lax.Precision.HIGHEST,
                        preferred_element_type=jnp.float32)
    h = h + lax.dot_general(x, wr_ref[...], dn,
                            precision=lax.Precision.HIGHEST,
                            preferred_element_type=jnp.float32)
    h = h + bl_ref[...][None, :]
    mu = jnp.mean(h, axis=0)
    var = jnp.mean((h - mu[None, :]) ** 2, axis=0)
    h = (h - mu[None, :]) * jax.lax.rsqrt(var + EPS) * g_ref[...][None, :] + b_ref[...][None, :]
    o_ref[...] = jnp.maximum(h, 0.0) + x


def kernel(x, edge_index, W_l, b_l, W_r, gamma, beta):
    src = edge_index[0].astype(jnp.int32)
    dst = edge_index[1].astype(jnp.int32)
    pad = E_PAD - N_EDGES
    src_p = jnp.concatenate([src, jnp.zeros((pad,), jnp.int32)]).reshape(E_PAD // _CH, _CH)
    dst_p = jnp.concatenate([dst, jnp.full((pad,), N_PAD - 1, jnp.int32)]).reshape(E_PAD // _CH, _CH)

    zeros2d = jnp.zeros((_CH, D), jnp.float32)
    zeros1d = jnp.zeros((_NODE_ROWS_PER_S,), jnp.float32)
    ones1d = jnp.ones((_CH,), jnp.float32)

    agg_p, deg_p = _make_sc_call()(src_p, dst_p, x, zeros2d, zeros1d, ones1d)

    return pl.pallas_call(
        _tc_body,
        out_shape=jax.ShapeDtypeStruct((N_NODES, D), jnp.float32),
    )(agg_p, deg_p, x, W_l, b_l, W_r, gamma, beta)


# speedup vs baseline: 1.0128x; 1.0128x over previous
import functools

import jax
import jax.numpy as jnp
from jax import lax
from jax.experimental import pallas as pl
from jax.experimental.pallas import tpu as pltpu
from jax.experimental.pallas import tpu_sc as plsc

N_NODES = 10000
N_PAD = 10240
N_EDGES = 320000
E_PAD = 327680
D = 128
EPS = 1e-5

_NC = 2
_NS = 16
_ROWS_PER_W = E_PAD // (_NC * _NS) // 128
_NODE_ROWS_PER_S = N_PAD // _NS
_NBUF = 2


def _sc_body(src_hbm, dst_hbm, x_hbm, zeros2d, zeros1d, ones1d,
             agg_out, deg_out,
             src_v, dst_v, gbufs, ones_v, zd_v, agg_sh, deg_sh,
             gsem0, gsem1, dsem):
    c = lax.axis_index("c")
    s = lax.axis_index("s")
    wid = c * _NS + s
    gsems = (gsem0, gsem1)

    pltpu.sync_copy(zeros2d, gbufs.at[0])
    pltpu.sync_copy(zeros1d, zd_v)
    for j in range(_NODE_ROWS_PER_S // 128):
        pltpu.sync_copy(gbufs.at[0], agg_sh.at[pl.ds(s * _NODE_ROWS_PER_S + j * 128, 128)])
    pltpu.sync_copy(zd_v, deg_sh.at[pl.ds(s * _NODE_ROWS_PER_S, _NODE_ROWS_PER_S)])

    base = wid * _ROWS_PER_W
    hrows = _ROWS_PER_W // 2
    pltpu.sync_copy(ones1d, ones_v)

    first = True
    for h in range(2):
        pltpu.sync_copy(src_hbm.at[pl.ds(base + h * hrows, hrows)], src_v)
        pltpu.sync_copy(dst_hbm.at[pl.ds(base + h * hrows, hrows)], dst_v)

        for b in range(_NBUF):
            pltpu.async_copy(x_hbm.at[src_v.at[b]], gbufs.at[b], gsems[b])

        if first:
            plsc.subcore_barrier()
            first = False

        def outer(i4, _):
            for b in range(_NBUF):
                i = i4 * _NBUF + b
                pltpu.make_async_copy(x_hbm.at[src_v.at[i]], gbufs.at[b], gsems[b]).wait()
                pltpu.sync_copy(gbufs.at[b], agg_sh.at[dst_v.at[i]], add=True)
                pltpu.sync_copy(ones_v, deg_sh.at[dst_v.at[i]], add=True)

                @pl.when(i + _NBUF < hrows)
                def _():
                    pltpu.async_copy(x_hbm.at[src_v.at[i + _NBUF]], gbufs.at[b], gsems[b])
            return ()

        lax.fori_loop(0, hrows // _NBUF, outer, (), unroll=False)

    plsc.subcore_barrier()

    nbase = s * _NODE_ROWS_PER_S
    for j in range(_NODE_ROWS_PER_S // 128):
        b = j % _NBUF
        pltpu.sync_copy(agg_sh.at[pl.ds(nbase + j * 128, 128)], gbufs.at[b])
        pltpu.sync_copy(gbufs.at[b], agg_out.at[c, pl.ds(nbase + j * 128, 128)])
    pltpu.sync_copy(deg_sh.at[pl.ds(nbase, _NODE_ROWS_PER_S)], zd_v)
    pltpu.sync_copy(zd_v, deg_out.at[c, pl.ds(nbase, _NODE_ROWS_PER_S)])


def _make_sc_call():
    return functools.partial(
        pl.kernel,
        mesh=plsc.VectorSubcoreMesh(core_axis_name="c", subcore_axis_name="s"),
        out_type=[
            jax.ShapeDtypeStruct((_NC, N_PAD, D), jnp.float32),
            jax.ShapeDtypeStruct((_NC, N_PAD), jnp.float32),
        ],
        scratch_types=[
            pltpu.VMEM((_ROWS_PER_W // 2, 128), jnp.int32),
            pltpu.VMEM((_ROWS_PER_W // 2, 128), jnp.int32),
            pltpu.VMEM((_NBUF, 128, D), jnp.float32),
            pltpu.VMEM((128,), jnp.float32),
            pltpu.VMEM((_NODE_ROWS_PER_S,), jnp.float32),
            pltpu.VMEM_SHARED((N_PAD, D), jnp.float32),
            pltpu.VMEM_SHARED((N_PAD,), jnp.float32),
            pltpu.SemaphoreType.DMA,
            pltpu.SemaphoreType.DMA,
            pltpu.SemaphoreType.DMA,
        ],
    )(_sc_body)


def _tc_body(a_ref, d_ref, x_ref, wl_ref, bl_ref, wr_ref, g_ref, b_ref, o_ref):
    agg = a_ref[0, :N_NODES, :] + a_ref[1, :N_NODES, :]
    deg = d_ref[0, :N_NODES] + d_ref[1, :N_NODES]
    deg = jnp.maximum(deg, 1.0)
    mean = agg / deg[:, None]
    x = x_ref[...]
    dn = (((1,), (1,)), ((), ()))
    h = lax.dot_general(mean, wl_ref[...], dn,
                        precision=lax.Precision.HIGHEST,
                        preferred_element_type=jnp.float32)
    h = h + lax.dot_general(x, wr_ref[...], dn,
                            precision=lax.Precision.HIGHEST,
                            preferred_element_type=jnp.float32)
    h = h + bl_ref[...][None, :]
    mu = jnp.mean(h, axis=0)
    var = jnp.mean((h - mu[None, :]) ** 2, axis=0)
    h = (h - mu[None, :]) * jax.lax.rsqrt(var + EPS) * g_ref[...][None, :] + b_ref[...][None, :]
    o_ref[...] = jnp.maximum(h, 0.0) + x


def kernel(x, edge_index, W_l, b_l, W_r, gamma, beta):
    src = edge_index[0].astype(jnp.int32)
    dst = edge_index[1].astype(jnp.int32)
    pad = E_PAD - N_EDGES
    src_p = jnp.concatenate([src, jnp.zeros((pad,), jnp.int32)]).reshape(E_PAD // 128, 128)
    dst_p = jnp.concatenate([dst, jnp.full((pad,), N_PAD - 1, jnp.int32)]).reshape(E_PAD // 128, 128)

    zeros2d = jnp.zeros((128, D), jnp.float32)
    zeros1d = jnp.zeros((_NODE_ROWS_PER_S,), jnp.float32)
    ones1d = jnp.ones((128,), jnp.float32)

    agg_p, deg_p = _make_sc_call()(src_p, dst_p, x, zeros2d, zeros1d, ones1d)

    return pl.pallas_call(
        _tc_body,
        out_shape=jax.ShapeDtypeStruct((N_NODES, D), jnp.float32),
    )(agg_p, deg_p, x, W_l, b_l, W_r, gamma, beta)
